# baseline (speedup 1.0000x reference)
import functools

import jax
import jax.numpy as jnp
from jax import lax
from jax.experimental import pallas as pl
from jax.experimental.pallas import tpu as pltpu
from jax.experimental.pallas import tpu_sc as plsc

H = 224
W = 224
C = 384
NPTS = 2
LANES = 16
BLK = 128
BLOCKS_PER_CH = (H * W) // BLK
CH_PER_WORKER = 16
N_WORKERS = C // CH_PER_WORKER
N_IDX = CH_PER_WORKER * 8


def _floor_i32(v):
    t = v.astype(jnp.int32)
    return jnp.where(t.astype(jnp.float32) > v, t - 1, t)


def _sc_body(x_hbm, grid_hbm, out_hbm, gridv, wv, lxv, bv, idxv, rowsv,
             stage, sem):
    info = plsc.get_sparse_core_info()
    nc = info.num_cores
    wid = lax.axis_index("s") * nc + lax.axis_index("c")

    @pl.when(wid < N_WORKERS)
    def _():
        pltpu.sync_copy(grid_hbm, gridv)
        lane = jnp.arange(16, dtype=jnp.int32)

        k = lane & 7
        p = k >> 2
        cy = (k >> 1) & 1
        cx = k & 1

        gx = plsc.load_gather(gridv, [2 * p])
        gy = plsc.load_gather(gridv, [2 * p + 1])

        ix = ((gx + 1.0) * W - 1.0) / 2.0
        iy = ((gy + 1.0) * H - 1.0) / 2.0
        ix = jnp.clip(ix, -2.0, float(W) + 1.0)
        iy = jnp.clip(iy, -2.0, float(H) + 1.0)
        x0 = _floor_i32(ix)
        y0 = _floor_i32(iy)
        fx = ix - x0.astype(jnp.float32)
        fy = iy - y0.astype(jnp.float32)

        xc = x0 + cx
        yc = y0 + cy
        wx = jnp.where(cx == 1, fx, 1.0 - fx)
        wy = jnp.where(cy == 1, fy, 1.0 - fy)
        valid = ((xc >= 0) & (xc <= W - 1) & (yc >= 0) & (yc <= H - 1))
        w = jnp.where(valid, wx * wy, 0.0)
        xi = jnp.clip(xc, 0, W - 1)
        yi = jnp.clip(yc, 0, H - 1)

        wv[...] = w
        pix = yi * W + xi
        lxv[...] = pix & (BLK - 1)
        bv[...] = pix >> 7

        j0 = wid * CH_PER_WORKER
        for t in range(N_IDX // 16):
            g_k = lane & 7
            g_j = t * 2 + (lane >> 3)
            bg = plsc.load_gather(bv, [g_k])
            idxv[pl.ds(t * 16, 16)] = (j0 + g_j) * BLOCKS_PER_CH + bg

        pltpu.async_copy(x_hbm.at[idxv], rowsv, sem).wait()

        for pt in range(NPTS):
            acc = jnp.zeros((16,), jnp.float32)
            for c4 in range(4):
                kk = pt * 4 + c4
                kvec = jnp.full((16,), kk, jnp.int32)
                w_k = plsc.load_gather(wv, [kvec])
                lx_k = plsc.load_gather(lxv, [kvec])
                vals = plsc.load_gather(rowsv, [lane * 8 + kk, lx_k])
                acc = acc + w_k * vals
            plsc.store_scatter(stage, [lane * NPTS + pt], acc)

        pltpu.sync_copy(stage, out_hbm.at[pl.ds(j0 * NPTS, CH_PER_WORKER * NPTS)])


_sc_call = functools.partial(
    pl.kernel,
    mesh=plsc.VectorSubcoreMesh(core_axis_name="c", subcore_axis_name="s"),
    compiler_params=pltpu.CompilerParams(needs_layout_passes=False),
    out_type=jax.ShapeDtypeStruct((C * NPTS,), jnp.float32),
    scratch_types=[
        pltpu.VMEM((16,), jnp.float32),
        pltpu.VMEM((16,), jnp.float32),
        pltpu.VMEM((16,), jnp.int32),
        pltpu.VMEM((16,), jnp.int32),
        pltpu.VMEM((N_IDX,), jnp.int32),
        pltpu.VMEM((N_IDX, BLK), jnp.float32),
        pltpu.VMEM((CH_PER_WORKER * NPTS,), jnp.float32),
        pltpu.SemaphoreType.DMA,
    ],
)(_sc_body)


@jax.jit
def kernel(x, grid):
    xv = x.reshape(-1, BLK)
    gpad = jnp.concatenate(
        [grid.reshape(-1), jnp.zeros((12,), jnp.float32)])
    out = _sc_call(xv, gpad)
    return out.reshape(1, C, 1, NPTS)


# speedup vs baseline: 2.3118x; 2.3118x over previous
import functools

import jax
import jax.numpy as jnp
from jax.experimental import pallas as pl
from jax.experimental.pallas import tpu as pltpu

H = 224
W = 224
C = 384
NPTS = 2


def _floor_f32(v):
    t = v.astype(jnp.int32)
    return jnp.where(t.astype(jnp.float32) > v, t - 1, t)


def _body(grid_ref, x_ref, out_ref, row00, row01, row10, row11, sem):
    rows = (row00, row01, row10, row11)

    copies = []
    pix = []
    for p in range(NPTS):
        gx = grid_ref[p, 0]
        gy = grid_ref[p, 1]
        ix = ((gx + 1.0) * W - 1.0) / 2.0
        iy = ((gy + 1.0) * H - 1.0) / 2.0
        ix = jnp.clip(ix, -2.0, float(W) + 1.0)
        iy = jnp.clip(iy, -2.0, float(H) + 1.0)
        x0 = _floor_f32(ix)
        y0 = _floor_f32(iy)
        fx = ix - x0.astype(jnp.float32)
        fy = iy - y0.astype(jnp.float32)
        xs = []
        for cx in range(2):
            xc = x0 + cx
            wxc = fx if cx == 1 else 1.0 - fx
            vx = ((xc >= 0) & (xc <= W - 1)).astype(jnp.float32)
            xs.append((jnp.clip(xc, 0, W - 1), wxc * vx))
        pix.append(xs)
        for cy in range(2):
            yc = y0 + cy
            wyc = fy if cy == 1 else 1.0 - fy
            vy = ((yc >= 0) & (yc <= H - 1)).astype(jnp.float32)
            yi = jnp.clip(yc, 0, H - 1)
            dst = rows[p * 2 + cy]
            cp = pltpu.make_async_copy(x_ref.at[:, yi, :], dst, sem)
            cp.start()
            copies.append((cp, wyc * vy))

    lanes = jax.lax.broadcasted_iota(jnp.int32, (1, W), 1)

    for p in range(NPTS):
        acc = jnp.zeros((C, 1), jnp.float32)
        for cy in range(2):
            cp, wy = copies[p * 2 + cy]
            cp.wait()
            band = rows[p * 2 + cy][...]
            for cx in range(2):
                xi, wx = pix[p][cx]
                col = jnp.where(lanes == xi, band, 0.0).sum(
                    axis=1, keepdims=True)
                acc = acc + col * (wx * wy)
        out_ref[:, pl.ds(p, 1)] = acc


_call = pl.pallas_call(
    _body,
    out_shape=jax.ShapeDtypeStruct((C, NPTS), jnp.float32),
    in_specs=[
        pl.BlockSpec(memory_space=pltpu.MemorySpace.SMEM),
        pl.BlockSpec(memory_space=pltpu.MemorySpace.HBM),
    ],
    out_specs=pl.BlockSpec(memory_space=pltpu.MemorySpace.VMEM),
    scratch_shapes=[
        pltpu.VMEM((C, W), jnp.float32),
        pltpu.VMEM((C, W), jnp.float32),
        pltpu.VMEM((C, W), jnp.float32),
        pltpu.VMEM((C, W), jnp.float32),
        pltpu.SemaphoreType.DMA,
    ],
)


@jax.jit
def kernel(x, grid):
    out = _call(grid.reshape(NPTS, 2), x.reshape(C, H, W))
    return out.reshape(1, C, 1, NPTS)
